# baseline (speedup 1.0000x reference)
import functools

import jax
import jax.numpy as jnp
from jax import lax
from jax.experimental import pallas as pl
from jax.experimental.pallas import tpu as pltpu
from jax.experimental.pallas import tpu_sc as plsc

N = 10000
E = 320000
H = 128
NC = 2
NS = 16
NW = NC * NS
PER_W = E // NW
C = 80
CH = PER_W // C
NPAD = 10240
ROWS_PER_TILE = NPAD // NS

BN_EPS = 1e-5

def _mk_mesh():
    return plsc.VectorSubcoreMesh(core_axis_name="c", subcore_axis_name="s",
                                  num_cores=NC, num_subcores=NS)


def _sc_msg_scatter(t, om, row, col, zeros_tile):
    @functools.partial(
        pl.kernel,
        out_type=jax.ShapeDtypeStruct((NC, NPAD, H), jnp.float32),
        mesh=_mk_mesh(),
        scratch_types=[
            pltpu.VMEM((C,), jnp.int32), pltpu.VMEM((C,), jnp.int32),
            pltpu.VMEM((C,), jnp.int32), pltpu.VMEM((C,), jnp.int32),
            pltpu.VMEM((C,), jnp.float32), pltpu.VMEM((C,), jnp.float32),
            pltpu.VMEM((C, H), jnp.float32), pltpu.VMEM((C, H), jnp.float32),
            pltpu.VMEM_SHARED((NPAD, H), jnp.float32),
            pltpu.SemaphoreType.DMA, pltpu.SemaphoreType.DMA,
            pltpu.SemaphoreType.DMA, pltpu.SemaphoreType.DMA,
        ],
    )
    def k(t_hbm, om_hbm, row_hbm, col_hbm, zero_hbm, p_hbm,
          rowv0, rowv1, colv0, colv1, omv0, omv1, rows0, rows1, acc,
          isem0, isem1, gsem0, gsem1):
        cid = lax.axis_index("c")
        sid = lax.axis_index("s")
        wid = sid * NC + cid
        base = wid * PER_W
        bufs = ((rowv0, colv0, omv0, rows0, isem0, gsem0),
                (rowv1, colv1, omv1, rows1, isem1, gsem1))

        pltpu.sync_copy(zero_hbm,
                        acc.at[pl.ds(sid * ROWS_PER_TILE, ROWS_PER_TILE)])

        def idx_start(g, b):
            off = base + g * C
            rowv, colv, omv, _, isem, _ = bufs[b]
            pltpu.async_copy(row_hbm.at[pl.ds(off, C)], rowv, isem)
            pltpu.async_copy(col_hbm.at[pl.ds(off, C)], colv, isem)
            pltpu.async_copy(om_hbm.at[pl.ds(off, C)], omv, isem)

        def idx_wait(b):
            rowv, colv, omv, _, isem, _ = bufs[b]
            pltpu.make_async_copy(row_hbm.at[pl.ds(0, C)], rowv, isem).wait()
            pltpu.make_async_copy(col_hbm.at[pl.ds(0, C)], colv, isem).wait()
            pltpu.make_async_copy(om_hbm.at[pl.ds(0, C)], omv, isem).wait()

        def gather_start(b):
            rowv, _, _, rows, _, gsem = bufs[b]
            pltpu.async_copy(t_hbm.at[rowv], rows, gsem)

        def gather_wait(b):
            rowv, _, _, rows, _, gsem = bufs[b]
            pltpu.make_async_copy(t_hbm.at[rowv], rows, gsem).wait()

        def scale(b):
            _, _, omv, rows, _, _ = bufs[b]

            def body(e16, carry):
                om16 = omv[pl.ds(e16 * 16, 16)]
                for l in range(16):
                    om_vec = lax.gather(
                        om16, jnp.full((16, 1), l, jnp.int32),
                        lax.GatherDimensionNumbers(offset_dims=(),
                                                   collapsed_slice_dims=(0,),
                                                   start_index_map=(0,)),
                        (1,), mode=lax.GatherScatterMode.PROMISE_IN_BOUNDS)
                    e = e16 * 16 + l
                    for j in range(8):
                        sl = pl.ds(j * 16, 16)
                        rows[e, sl] = rows[e, sl] * om_vec
                return carry

            lax.fori_loop(0, C // 16, body, 0)

        idx_start(0, 0)
        plsc.subcore_barrier()
        idx_wait(0)
        gather_start(0)
        idx_start(1, 1)

        @pl.loop(0, CH, step=2)
        def _outer(g0):
            for b in range(2):
                g = g0 + b

                @pl.when(g < CH)
                def _():
                    _, colv, _, rows, _, _ = bufs[b]
                    gather_wait(b)

                    @pl.when(g + 1 < CH)
                    def _():
                        idx_wait(1 - b)
                        gather_start(1 - b)

                    scale(b)
                    pltpu.sync_copy(rows, acc.at[colv], add=True)

                    @pl.when(g + 2 < CH)
                    def _():
                        idx_start(g + 2, b)

        plsc.subcore_barrier()
        pltpu.sync_copy(
            acc.at[pl.ds(sid * ROWS_PER_TILE, ROWS_PER_TILE)],
            p_hbm.at[cid].at[pl.ds(sid * ROWS_PER_TILE, ROWS_PER_TILE)])

    return k(t, om, row, col, zeros_tile)


def _sc_gather2(xn, row, col):
    @functools.partial(
        pl.kernel,
        out_type=(
            jax.ShapeDtypeStruct((E, H), jnp.float32),
            jax.ShapeDtypeStruct((E, H), jnp.float32),
        ),
        mesh=_mk_mesh(),
        scratch_types=[
            pltpu.VMEM((C,), jnp.int32), pltpu.VMEM((C,), jnp.int32),
            pltpu.VMEM((C,), jnp.int32), pltpu.VMEM((C,), jnp.int32),
            pltpu.VMEM((C, H), jnp.float32), pltpu.VMEM((C, H), jnp.float32),
            pltpu.VMEM((C, H), jnp.float32), pltpu.VMEM((C, H), jnp.float32),
            pltpu.SemaphoreType.DMA, pltpu.SemaphoreType.DMA,
            pltpu.SemaphoreType.DMA, pltpu.SemaphoreType.DMA,
        ],
    )
    def k(xn_hbm, row_hbm, col_hbm, src_hbm, tgt_hbm,
          rowv0, rowv1, colv0, colv1, sb0, sb1, tb0, tb1,
          isem0, isem1, gsem0, gsem1):
        wid = lax.axis_index("s") * NC + lax.axis_index("c")
        base = wid * PER_W
        bufs = ((rowv0, colv0, sb0, tb0, isem0, gsem0),
                (rowv1, colv1, sb1, tb1, isem1, gsem1))

        def idx_start(g, b):
            off = base + g * C
            rowv, colv, _, _, isem, _ = bufs[b]
            pltpu.async_copy(row_hbm.at[pl.ds(off, C)], rowv, isem)
            pltpu.async_copy(col_hbm.at[pl.ds(off, C)], colv, isem)

        def idx_wait(b):
            rowv, colv, _, _, isem, _ = bufs[b]
            pltpu.make_async_copy(row_hbm.at[pl.ds(0, C)], rowv, isem).wait()
            pltpu.make_async_copy(col_hbm.at[pl.ds(0, C)], colv, isem).wait()

        def gather_start(b):
            rowv, colv, sb, tb, _, gsem = bufs[b]
            pltpu.async_copy(xn_hbm.at[rowv], sb, gsem)
            pltpu.async_copy(xn_hbm.at[colv], tb, gsem)

        def gather_wait(b):
            rowv, colv, sb, tb, _, gsem = bufs[b]
            pltpu.make_async_copy(xn_hbm.at[rowv], sb, gsem).wait()
            pltpu.make_async_copy(xn_hbm.at[colv], tb, gsem).wait()

        idx_start(0, 0)
        idx_wait(0)
        gather_start(0)
        idx_start(1, 1)

        @pl.loop(0, CH, step=2)
        def _outer(g0):
            for b in range(2):
                g = g0 + b

                @pl.when(g < CH)
                def _():
                    _, _, sb, tb, _, _ = bufs[b]
                    gather_wait(b)

                    @pl.when(g + 1 < CH)
                    def _():
                        idx_wait(1 - b)
                        gather_start(1 - b)

                    @pl.when(g + 2 < CH)
                    def _():
                        idx_start(g + 2, b)

                    off = base + g * C
                    pltpu.sync_copy(sb, src_hbm.at[pl.ds(off, C)])
                    pltpu.sync_copy(tb, tgt_hbm.at[pl.ds(off, C)])

    return k(xn, row, col)


BN_TILE = 1000
BE = 2000


def _relu(v):
    return jnp.maximum(v, 0.0)


def _dot(a, b):
    return jnp.dot(a, b, preferred_element_type=jnp.float32)


def _tc_node_encode(x, npw, npb, pnw, pnb):
    def k(x_ref, npw_ref, npb_ref, pnw_ref, pnb_ref, x0_ref, t1_ref):
        x0 = _relu(_dot(x_ref[...], npw_ref[...]) + npb_ref[...])
        x0_ref[...] = x0
        t1_ref[...] = _relu(_dot(x0, pnw_ref[...]) + pnb_ref[...])

    g = N // BN_TILE
    return pl.pallas_call(
        k,
        grid=(g,),
        in_specs=[
            pl.BlockSpec((BN_TILE, H), lambda i: (i, 0)),
            pl.BlockSpec((H, H), lambda i: (0, 0)),
            pl.BlockSpec((1, H), lambda i: (0, 0)),
            pl.BlockSpec((H, H), lambda i: (0, 0)),
            pl.BlockSpec((1, H), lambda i: (0, 0)),
        ],
        out_specs=[
            pl.BlockSpec((BN_TILE, H), lambda i: (i, 0)),
            pl.BlockSpec((BN_TILE, H), lambda i: (i, 0)),
        ],
        out_shape=[
            jax.ShapeDtypeStruct((N, H), jnp.float32),
            jax.ShapeDtypeStruct((N, H), jnp.float32),
        ],
    )(x, npw, npb, pnw, pnb)


def _tc_edge_encode(edge_attr, epw, epb):
    def k(ea_ref, w_ref, b_ref, out_ref):
        out_ref[...] = _relu(_dot(ea_ref[...], w_ref[...]) + b_ref[...])

    g = E // BE
    d_edge = edge_attr.shape[1]
    return pl.pallas_call(
        k,
        grid=(g,),
        in_specs=[
            pl.BlockSpec((BE, d_edge), lambda i: (i, 0)),
            pl.BlockSpec((d_edge, H), lambda i: (0, 0)),
            pl.BlockSpec((1, H), lambda i: (0, 0)),
        ],
        out_specs=pl.BlockSpec((BE, H), lambda i: (i, 0)),
        out_shape=jax.ShapeDtypeStruct((E, H), jnp.float32),
    )(edge_attr, epw, epb)


def _dotg_t(a, b):
    return lax.dot_general(a, b, (((0,), (1,)), ((), ())),
                           preferred_element_type=jnp.float32)


def _tc_omega1(ea0, pw1, pb1c, pw2r, pb2):
    def k(ea_ref, w1_ref, b1_ref, w2_ref, b2_ref, om_ref):
        hT = _relu(_dotg_t(w1_ref[...], ea_ref[...]) + b1_ref[...])
        om = _dot(w2_ref[...], hT) + b2_ref[...]
        om_ref[...] = om[None]

    g = E // BE
    return pl.pallas_call(
        k,
        grid=(g,),
        in_specs=[
            pl.BlockSpec((BE, H), lambda i: (i, 0)),
            pl.BlockSpec((H, 32), lambda i: (0, 0)),
            pl.BlockSpec((32, 1), lambda i: (0, 0)),
            pl.BlockSpec((1, 32), lambda i: (0, 0)),
            pl.BlockSpec((1, 1), lambda i: (0, 0)),
        ],
        out_specs=pl.BlockSpec((1, 1, BE), lambda i: (i, 0, 0)),
        out_shape=jax.ShapeDtypeStruct((g, 1, BE), jnp.float32),
    )(ea0, pw1, pb1c, pw2r, pb2)


def _tc_sum_stats(xin, partials):
    def k(x_ref, p_ref, s_ref, st_ref):
        s = x_ref[...] + p_ref[0] + p_ref[1]
        s_ref[...] = s
        ones_row = jnp.ones((1, BN_TILE), jnp.float32)
        upd = jnp.concatenate(
            [_dot(ones_row, s), _dot(ones_row, s * s),
             jnp.zeros((6, H), jnp.float32)], axis=0)

        @pl.when(pl.program_id(0) == 0)
        def _():
            st_ref[...] = jnp.zeros_like(st_ref)

        st_ref[...] += upd

    g = N // BN_TILE
    return pl.pallas_call(
        k,
        grid=(g,),
        in_specs=[
            pl.BlockSpec((BN_TILE, H), lambda i: (i, 0)),
            pl.BlockSpec((NC, BN_TILE, H), lambda i: (0, i, 0)),
        ],
        out_specs=[
            pl.BlockSpec((BN_TILE, H), lambda i: (i, 0)),
            pl.BlockSpec((8, H), lambda i: (0, 0)),
        ],
        out_shape=[
            jax.ShapeDtypeStruct((N, H), jnp.float32),
            jax.ShapeDtypeStruct((8, H), jnp.float32),
        ],
    )(xin, partials)


def _tc_bn_relu_node(s, stats, gamma, beta, pnw=None, pnb=None):
    with_t = pnw is not None

    def k(*refs):
        if with_t:
            s_ref, st_ref, g_ref, b_ref, w_ref, wb_ref, xn_ref, t_ref = refs
        else:
            s_ref, st_ref, g_ref, b_ref, xn_ref = refs
        st = st_ref[...]
        mu = st[0:1] * (1.0 / N)
        var = st[1:2] * (1.0 / N) - mu * mu
        xn = _relu(g_ref[...] * (s_ref[...] - mu) * lax.rsqrt(var + BN_EPS)
                   + b_ref[...])
        xn_ref[...] = xn
        if with_t:
            t_ref[...] = _relu(_dot(xn, w_ref[...]) + wb_ref[...])

    g = N // BN_TILE
    in_specs = [
        pl.BlockSpec((BN_TILE, H), lambda i: (i, 0)),
        pl.BlockSpec((8, H), lambda i: (0, 0)),
        pl.BlockSpec((1, H), lambda i: (0, 0)),
        pl.BlockSpec((1, H), lambda i: (0, 0)),
    ]
    args = [s, stats, gamma, beta]
    out_specs = [pl.BlockSpec((BN_TILE, H), lambda i: (i, 0))]
    out_shape = [jax.ShapeDtypeStruct((N, H), jnp.float32)]
    if with_t:
        in_specs += [pl.BlockSpec((H, H), lambda i: (0, 0)),
                     pl.BlockSpec((1, H), lambda i: (0, 0))]
        args += [pnw, pnb]
        out_specs.append(pl.BlockSpec((BN_TILE, H), lambda i: (i, 0)))
        out_shape.append(jax.ShapeDtypeStruct((N, H), jnp.float32))
    res = pl.pallas_call(
        k, grid=(g,), in_specs=in_specs, out_specs=out_specs,
        out_shape=out_shape,
    )(*args)
    return res if with_t else res[0]


def _tc_edge_mlp(ea0, src, tgt, w_ea, w_src, w_tgt, w_ds, w_cos, b1, w2, b2):
    def k(ea_ref, s_ref, t_ref, wea_ref, wsrc_ref, wtgt_ref, wds_ref,
          wcos_ref, b1_ref, w2_ref, b2_ref, y_ref, st_ref):
        s = s_ref[...]
        t = t_ref[...]
        d = jnp.abs(s - t)
        ones_col = jnp.ones((H, 1), jnp.float32)
        sn2 = _dot(s * s, ones_col)
        tn2 = _dot(t * t, ones_col)
        st = _dot(s * t, ones_col)
        cos = st / jnp.maximum(jnp.sqrt(sn2 * tn2), 1e-8)
        h = _relu(_dot(ea_ref[...], wea_ref[...]) + _dot(s, wsrc_ref[...])
                  + _dot(t, wtgt_ref[...]) + _dot(d, wds_ref[...])
                  + cos * wcos_ref[...] + b1_ref[...])
        y = _dot(h, w2_ref[...]) + b2_ref[...]
        y_ref[...] = y
        ones_row = jnp.ones((1, BE), jnp.float32)
        upd = jnp.concatenate(
            [_dot(ones_row, y), _dot(ones_row, y * y),
             jnp.zeros((6, H), jnp.float32)], axis=0)

        @pl.when(pl.program_id(0) == 0)
        def _():
            st_ref[...] = jnp.zeros_like(st_ref)

        st_ref[...] += upd

    g = E // BE
    return pl.pallas_call(
        k,
        grid=(g,),
        in_specs=[
            pl.BlockSpec((BE, H), lambda i: (i, 0)),
            pl.BlockSpec((BE, H), lambda i: (i, 0)),
            pl.BlockSpec((BE, H), lambda i: (i, 0)),
            pl.BlockSpec((H, H), lambda i: (0, 0)),
            pl.BlockSpec((H, H), lambda i: (0, 0)),
            pl.BlockSpec((H, H), lambda i: (0, 0)),
            pl.BlockSpec((H, H), lambda i: (0, 0)),
            pl.BlockSpec((1, H), lambda i: (0, 0)),
            pl.BlockSpec((1, H), lambda i: (0, 0)),
            pl.BlockSpec((H, H), lambda i: (0, 0)),
            pl.BlockSpec((1, H), lambda i: (0, 0)),
        ],
        out_specs=[
            pl.BlockSpec((BE, H), lambda i: (i, 0)),
            pl.BlockSpec((8, H), lambda i: (0, 0)),
        ],
        out_shape=[
            jax.ShapeDtypeStruct((E, H), jnp.float32),
            jax.ShapeDtypeStruct((8, H), jnp.float32),
        ],
    )(ea0, src, tgt, w_ea, w_src, w_tgt, w_ds, w_cos, b1, w2, b2)


def _tc_omega2(y, stats, gamma, beta, pw1, pb1c, pw2r, pb2):
    def k(y_ref, st_ref, g_ref, b_ref, w1_ref, b1_ref, w2_ref, b2_ref,
          om_ref):
        st = st_ref[...]
        mu = st[0:1] * (1.0 / E)
        var = st[1:2] * (1.0 / E) - mu * mu
        ea1 = _relu(g_ref[...] * (y_ref[...] - mu) * lax.rsqrt(var + BN_EPS)
                    + b_ref[...])
        hT = _relu(_dotg_t(w1_ref[...], ea1) + b1_ref[...])
        om = _dot(w2_ref[...], hT) + b2_ref[...]
        om_ref[...] = om[None]

    g = E // BE
    return pl.pallas_call(
        k,
        grid=(g,),
        in_specs=[
            pl.BlockSpec((BE, H), lambda i: (i, 0)),
            pl.BlockSpec((8, H), lambda i: (0, 0)),
            pl.BlockSpec((1, H), lambda i: (0, 0)),
            pl.BlockSpec((1, H), lambda i: (0, 0)),
            pl.BlockSpec((H, 32), lambda i: (0, 0)),
            pl.BlockSpec((32, 1), lambda i: (0, 0)),
            pl.BlockSpec((1, 32), lambda i: (0, 0)),
            pl.BlockSpec((1, 1), lambda i: (0, 0)),
        ],
        out_specs=pl.BlockSpec((1, 1, BE), lambda i: (i, 0, 0)),
        out_shape=jax.ShapeDtypeStruct((g, 1, BE), jnp.float32),
    )(y, stats, gamma, beta, pw1, pb1c, pw2r, pb2)


def kernel(x, edge_index, edge_attr, params):
    row = edge_index[0]
    col = edge_index[1]
    p = params
    b0, b1 = p['blocks'][0], p['blocks'][1]

    def r2(v):
        return v.reshape(1, -1)

    zeros_tile = jnp.zeros((ROWS_PER_TILE, H), jnp.float32)

    x0, t1 = _tc_node_encode(x, p['np_w'], r2(p['np_b']),
                             b0['pn_w'], r2(b0['pn_b']))
    ea0 = _tc_edge_encode(edge_attr, p['ep_w'], r2(p['ep_b']))

    om1 = _tc_omega1(ea0, b0['pe_w1'], b0['pe_b1'].reshape(32, 1),
                     b0['pe_w2'].reshape(1, 32), b0['pe_b2'].reshape(1, 1))
    p1 = _sc_msg_scatter(t1, om1.reshape(E), row, col, zeros_tile)
    s1, st1 = _tc_sum_stats(x0, p1)
    xn1, t2 = _tc_bn_relu_node(s1, st1, r2(b0['bn_ng']), r2(b0['bn_nb']),
                               b1['pn_w'], r2(b1['pn_b']))

    src, tgt = _sc_gather2(xn1, row, col)
    ee_w1 = b0['ee_w1']
    y, ste = _tc_edge_mlp(
        ea0, src, tgt,
        ee_w1[0:H], ee_w1[H:2 * H], ee_w1[2 * H:3 * H], ee_w1[3 * H:4 * H],
        ee_w1[4 * H:4 * H + 1], r2(b0['ee_b1']), b0['ee_w2'], r2(b0['ee_b2']))

    om2 = _tc_omega2(y, ste, r2(b0['bn_eg']), r2(b0['bn_eb']),
                     b1['pe_w1'], b1['pe_b1'].reshape(32, 1),
                     b1['pe_w2'].reshape(1, 32), b1['pe_b2'].reshape(1, 1))
    p2 = _sc_msg_scatter(t2, om2.reshape(E), row, col, zeros_tile)
    s2, st2 = _tc_sum_stats(xn1, p2)
    xn2 = _tc_bn_relu_node(s2, st2, r2(b1['bn_ng']), r2(b1['bn_nb']))
    return xn2


# speedup vs baseline: 1.1089x; 1.1089x over previous
import functools

import jax
import jax.numpy as jnp
from jax import lax
from jax.experimental import pallas as pl
from jax.experimental.pallas import tpu as pltpu
from jax.experimental.pallas import tpu_sc as plsc

N = 10000
E = 320000
H = 128
NC = 2
NS = 16
NW = NC * NS
PER_W = E // NW
C = 80
CH = PER_W // C
NPAD = 10240
ROWS_PER_TILE = NPAD // NS

BN_EPS = 1e-5

def _mk_mesh():
    return plsc.VectorSubcoreMesh(core_axis_name="c", subcore_axis_name="s",
                                  num_cores=NC, num_subcores=NS)


def _sc_msg_scatter(t, om, row, col, zeros_tile):
    @functools.partial(
        pl.kernel,
        out_type=jax.ShapeDtypeStruct((NC, NPAD, H), jnp.float32),
        mesh=_mk_mesh(),
        scratch_types=[
            pltpu.VMEM((C,), jnp.int32), pltpu.VMEM((C,), jnp.int32),
            pltpu.VMEM((C,), jnp.int32), pltpu.VMEM((C,), jnp.int32),
            pltpu.VMEM((C,), jnp.float32), pltpu.VMEM((C,), jnp.float32),
            pltpu.VMEM((C, H), jnp.float32), pltpu.VMEM((C, H), jnp.float32),
            pltpu.VMEM_SHARED((NPAD, H), jnp.float32),
            pltpu.SemaphoreType.DMA, pltpu.SemaphoreType.DMA,
            pltpu.SemaphoreType.DMA, pltpu.SemaphoreType.DMA,
        ],
    )
    def k(t_hbm, om_hbm, row_hbm, col_hbm, zero_hbm, p_hbm,
          rowv0, rowv1, colv0, colv1, omv0, omv1, rows0, rows1, acc,
          isem0, isem1, gsem0, gsem1):
        cid = lax.axis_index("c")
        sid = lax.axis_index("s")
        wid = sid * NC + cid
        base = wid * PER_W
        bufs = ((rowv0, colv0, omv0, rows0, isem0, gsem0),
                (rowv1, colv1, omv1, rows1, isem1, gsem1))

        pltpu.sync_copy(zero_hbm,
                        acc.at[pl.ds(sid * ROWS_PER_TILE, ROWS_PER_TILE)])

        def idx_start(g, b):
            off = base + g * C
            rowv, colv, omv, _, isem, _ = bufs[b]
            pltpu.async_copy(row_hbm.at[pl.ds(off, C)], rowv, isem)
            pltpu.async_copy(col_hbm.at[pl.ds(off, C)], colv, isem)
            pltpu.async_copy(om_hbm.at[pl.ds(off, C)], omv, isem)

        def idx_wait(b):
            rowv, colv, omv, _, isem, _ = bufs[b]
            pltpu.make_async_copy(row_hbm.at[pl.ds(0, C)], rowv, isem).wait()
            pltpu.make_async_copy(col_hbm.at[pl.ds(0, C)], colv, isem).wait()
            pltpu.make_async_copy(om_hbm.at[pl.ds(0, C)], omv, isem).wait()

        def gather_start(b):
            rowv, _, _, rows, _, gsem = bufs[b]
            pltpu.async_copy(t_hbm.at[rowv], rows, gsem)

        def gather_wait(b):
            rowv, _, _, rows, _, gsem = bufs[b]
            pltpu.make_async_copy(t_hbm.at[rowv], rows, gsem).wait()

        def scale(b):
            _, _, omv, rows, _, _ = bufs[b]

            def body(e16, carry):
                om16 = omv[pl.ds(e16 * 16, 16)]
                for l in range(16):
                    om_vec = lax.gather(
                        om16, jnp.full((16, 1), l, jnp.int32),
                        lax.GatherDimensionNumbers(offset_dims=(),
                                                   collapsed_slice_dims=(0,),
                                                   start_index_map=(0,)),
                        (1,), mode=lax.GatherScatterMode.PROMISE_IN_BOUNDS)
                    e = e16 * 16 + l
                    for j in range(8):
                        sl = pl.ds(j * 16, 16)
                        rows[e, sl] = rows[e, sl] * om_vec
                return carry

            lax.fori_loop(0, C // 16, body, 0)

        idx_start(0, 0)
        plsc.subcore_barrier()
        idx_wait(0)
        gather_start(0)
        idx_start(1, 1)

        @pl.loop(0, CH, step=2)
        def _outer(g0):
            for b in range(2):
                g = g0 + b

                @pl.when(g < CH)
                def _():
                    _, colv, _, rows, _, _ = bufs[b]
                    gather_wait(b)

                    @pl.when(g + 1 < CH)
                    def _():
                        idx_wait(1 - b)
                        gather_start(1 - b)

                    scale(b)
                    pltpu.sync_copy(rows, acc.at[colv], add=True)

                    @pl.when(g + 2 < CH)
                    def _():
                        idx_start(g + 2, b)

        plsc.subcore_barrier()
        pltpu.sync_copy(
            acc.at[pl.ds(sid * ROWS_PER_TILE, ROWS_PER_TILE)],
            p_hbm.at[cid].at[pl.ds(sid * ROWS_PER_TILE, ROWS_PER_TILE)])

    return k(t, om, row, col, zeros_tile)


def _sc_gather2(xn, row, col, e_off, e_num):
    per_w = e_num // NW
    n_ch = per_w // C

    @functools.partial(
        pl.kernel,
        out_type=(
            jax.ShapeDtypeStruct((e_num, H), jnp.float32),
            jax.ShapeDtypeStruct((e_num, H), jnp.float32),
        ),
        mesh=_mk_mesh(),
        scratch_types=[
            pltpu.VMEM((C,), jnp.int32), pltpu.VMEM((C,), jnp.int32),
            pltpu.VMEM((C,), jnp.int32), pltpu.VMEM((C,), jnp.int32),
            pltpu.VMEM((C, H), jnp.float32), pltpu.VMEM((C, H), jnp.float32),
            pltpu.VMEM((C, H), jnp.float32), pltpu.VMEM((C, H), jnp.float32),
            pltpu.SemaphoreType.DMA, pltpu.SemaphoreType.DMA,
            pltpu.SemaphoreType.DMA, pltpu.SemaphoreType.DMA,
        ],
    )
    def k(xn_hbm, row_hbm, col_hbm, src_hbm, tgt_hbm,
          rowv0, rowv1, colv0, colv1, sb0, sb1, tb0, tb1,
          isem0, isem1, gsem0, gsem1):
        wid = lax.axis_index("s") * NC + lax.axis_index("c")
        base = wid * per_w

        bufs = ((rowv0, colv0, sb0, tb0, isem0, gsem0),
                (rowv1, colv1, sb1, tb1, isem1, gsem1))

        def idx_start(g, b):
            off = e_off + base + g * C
            rowv, colv, _, _, isem, _ = bufs[b]
            pltpu.async_copy(row_hbm.at[pl.ds(off, C)], rowv, isem)
            pltpu.async_copy(col_hbm.at[pl.ds(off, C)], colv, isem)

        def idx_wait(b):
            rowv, colv, _, _, isem, _ = bufs[b]
            pltpu.make_async_copy(row_hbm.at[pl.ds(0, C)], rowv, isem).wait()
            pltpu.make_async_copy(col_hbm.at[pl.ds(0, C)], colv, isem).wait()

        def gather_start(b):
            rowv, colv, sb, tb, _, gsem = bufs[b]
            pltpu.async_copy(xn_hbm.at[rowv], sb, gsem)
            pltpu.async_copy(xn_hbm.at[colv], tb, gsem)

        def gather_wait(b):
            rowv, colv, sb, tb, _, gsem = bufs[b]
            pltpu.make_async_copy(xn_hbm.at[rowv], sb, gsem).wait()
            pltpu.make_async_copy(xn_hbm.at[colv], tb, gsem).wait()

        idx_start(0, 0)
        idx_wait(0)
        gather_start(0)
        idx_start(1, 1)

        @pl.loop(0, n_ch, step=2)
        def _outer(g0):
            for b in range(2):
                g = g0 + b

                @pl.when(g < n_ch)
                def _():
                    _, _, sb, tb, _, _ = bufs[b]
                    gather_wait(b)

                    @pl.when(g + 1 < n_ch)
                    def _():
                        idx_wait(1 - b)
                        gather_start(1 - b)

                    @pl.when(g + 2 < n_ch)
                    def _():
                        idx_start(g + 2, b)

                    off = base + g * C
                    pltpu.sync_copy(sb, src_hbm.at[pl.ds(off, C)])
                    pltpu.sync_copy(tb, tgt_hbm.at[pl.ds(off, C)])

    return k(xn, row, col)


BN_TILE = 1000
BE = 2000


def _relu(v):
    return jnp.maximum(v, 0.0)


def _dot(a, b):
    return jnp.dot(a, b, preferred_element_type=jnp.float32)


def _tc_node_encode(x, npw, npb, pnw, pnb):
    def k(x_ref, npw_ref, npb_ref, pnw_ref, pnb_ref, x0_ref, t1_ref):
        x0 = _relu(_dot(x_ref[...], npw_ref[...]) + npb_ref[...])
        x0_ref[...] = x0
        t1_ref[...] = _relu(_dot(x0, pnw_ref[...]) + pnb_ref[...])

    g = N // BN_TILE
    return pl.pallas_call(
        k,
        grid=(g,),
        in_specs=[
            pl.BlockSpec((BN_TILE, H), lambda i: (i, 0)),
            pl.BlockSpec((H, H), lambda i: (0, 0)),
            pl.BlockSpec((1, H), lambda i: (0, 0)),
            pl.BlockSpec((H, H), lambda i: (0, 0)),
            pl.BlockSpec((1, H), lambda i: (0, 0)),
        ],
        out_specs=[
            pl.BlockSpec((BN_TILE, H), lambda i: (i, 0)),
            pl.BlockSpec((BN_TILE, H), lambda i: (i, 0)),
        ],
        out_shape=[
            jax.ShapeDtypeStruct((N, H), jnp.float32),
            jax.ShapeDtypeStruct((N, H), jnp.float32),
        ],
    )(x, npw, npb, pnw, pnb)


def _tc_edge_enc_om(edge_attr, epw, epb, pw1, pb1c, pw2r, pb2):
    def k(ea_ref, w_ref, b_ref, w1_ref, b1_ref, w2_ref, b2_ref,
          out_ref, om_ref):
        ea0 = _relu(_dot(ea_ref[...], w_ref[...]) + b_ref[...])
        out_ref[...] = ea0
        hT = _relu(_dotg_t(w1_ref[...], ea0) + b1_ref[...])
        om = _dot(w2_ref[...], hT) + b2_ref[...]
        om_ref[...] = om[None]

    g = E // BE
    d_edge = edge_attr.shape[1]
    return pl.pallas_call(
        k,
        grid=(g,),
        in_specs=[
            pl.BlockSpec((BE, d_edge), lambda i: (i, 0)),
            pl.BlockSpec((d_edge, H), lambda i: (0, 0)),
            pl.BlockSpec((1, H), lambda i: (0, 0)),
            pl.BlockSpec((H, 32), lambda i: (0, 0)),
            pl.BlockSpec((32, 1), lambda i: (0, 0)),
            pl.BlockSpec((1, 32), lambda i: (0, 0)),
            pl.BlockSpec((1, 1), lambda i: (0, 0)),
        ],
        out_specs=[
            pl.BlockSpec((BE, H), lambda i: (i, 0)),
            pl.BlockSpec((1, 1, BE), lambda i: (i, 0, 0)),
        ],
        out_shape=[
            jax.ShapeDtypeStruct((E, H), jnp.float32),
            jax.ShapeDtypeStruct((g, 1, BE), jnp.float32),
        ],
    )(edge_attr, epw, epb, pw1, pb1c, pw2r, pb2)


def _dotg_t(a, b):
    return lax.dot_general(a, b, (((0,), (1,)), ((), ())),
                           preferred_element_type=jnp.float32)


def _tc_sum_stats(xin, partials):
    def k(x_ref, p_ref, s_ref, st_ref):
        s = x_ref[...] + p_ref[0] + p_ref[1]
        s_ref[...] = s
        ones_row = jnp.ones((1, BN_TILE), jnp.float32)
        upd = jnp.concatenate(
            [_dot(ones_row, s), _dot(ones_row, s * s),
             jnp.zeros((6, H), jnp.float32)], axis=0)

        @pl.when(pl.program_id(0) == 0)
        def _():
            st_ref[...] = jnp.zeros_like(st_ref)

        st_ref[...] += upd

    g = N // BN_TILE
    return pl.pallas_call(
        k,
        grid=(g,),
        in_specs=[
            pl.BlockSpec((BN_TILE, H), lambda i: (i, 0)),
            pl.BlockSpec((NC, BN_TILE, H), lambda i: (0, i, 0)),
        ],
        out_specs=[
            pl.BlockSpec((BN_TILE, H), lambda i: (i, 0)),
            pl.BlockSpec((8, H), lambda i: (0, 0)),
        ],
        out_shape=[
            jax.ShapeDtypeStruct((N, H), jnp.float32),
            jax.ShapeDtypeStruct((8, H), jnp.float32),
        ],
    )(xin, partials)


def _tc_bn_relu_node(s, stats, gamma, beta, pnw=None, pnb=None):
    with_t = pnw is not None

    def k(*refs):
        if with_t:
            s_ref, st_ref, g_ref, b_ref, w_ref, wb_ref, xn_ref, t_ref = refs
        else:
            s_ref, st_ref, g_ref, b_ref, xn_ref = refs
        st = st_ref[...]
        mu = st[0:1] * (1.0 / N)
        var = st[1:2] * (1.0 / N) - mu * mu
        xn = _relu(g_ref[...] * (s_ref[...] - mu) * lax.rsqrt(var + BN_EPS)
                   + b_ref[...])
        xn_ref[...] = xn
        if with_t:
            t_ref[...] = _relu(_dot(xn, w_ref[...]) + wb_ref[...])

    g = N // BN_TILE
    in_specs = [
        pl.BlockSpec((BN_TILE, H), lambda i: (i, 0)),
        pl.BlockSpec((8, H), lambda i: (0, 0)),
        pl.BlockSpec((1, H), lambda i: (0, 0)),
        pl.BlockSpec((1, H), lambda i: (0, 0)),
    ]
    args = [s, stats, gamma, beta]
    out_specs = [pl.BlockSpec((BN_TILE, H), lambda i: (i, 0))]
    out_shape = [jax.ShapeDtypeStruct((N, H), jnp.float32)]
    if with_t:
        in_specs += [pl.BlockSpec((H, H), lambda i: (0, 0)),
                     pl.BlockSpec((1, H), lambda i: (0, 0))]
        args += [pnw, pnb]
        out_specs.append(pl.BlockSpec((BN_TILE, H), lambda i: (i, 0)))
        out_shape.append(jax.ShapeDtypeStruct((N, H), jnp.float32))
    res = pl.pallas_call(
        k, grid=(g,), in_specs=in_specs, out_specs=out_specs,
        out_shape=out_shape,
    )(*args)
    return res if with_t else res[0]


def _tc_edge_mlp(ea0, src, tgt, w_ea, w_src, w_tgt, w_ds, w_cos, b1, w2, b2,
                 e_off, e_num):
    def k(ea_ref, s_ref, t_ref, wea_ref, wsrc_ref, wtgt_ref, wds_ref,
          wcos_ref, b1_ref, w2_ref, b2_ref, y_ref, st_ref):
        s = s_ref[...]
        t = t_ref[...]
        d = jnp.abs(s - t)
        ones_col = jnp.ones((H, 1), jnp.float32)
        sn2 = _dot(s * s, ones_col)
        tn2 = _dot(t * t, ones_col)
        st = _dot(s * t, ones_col)
        cos = st / jnp.maximum(jnp.sqrt(sn2 * tn2), 1e-8)
        h = _relu(_dot(ea_ref[...], wea_ref[...]) + _dot(s, wsrc_ref[...])
                  + _dot(t, wtgt_ref[...]) + _dot(d, wds_ref[...])
                  + cos * wcos_ref[...] + b1_ref[...])
        y = _dot(h, w2_ref[...]) + b2_ref[...]
        y_ref[...] = y
        ones_row = jnp.ones((1, BE), jnp.float32)
        upd = jnp.concatenate(
            [_dot(ones_row, y), _dot(ones_row, y * y),
             jnp.zeros((6, H), jnp.float32)], axis=0)

        @pl.when(pl.program_id(0) == 0)
        def _():
            st_ref[...] = jnp.zeros_like(st_ref)

        st_ref[...] += upd

    g = e_num // BE
    blk0 = e_off // BE
    return pl.pallas_call(
        k,
        grid=(g,),
        in_specs=[
            pl.BlockSpec((BE, H), lambda i: (i + blk0, 0)),
            pl.BlockSpec((BE, H), lambda i: (i, 0)),
            pl.BlockSpec((BE, H), lambda i: (i, 0)),
            pl.BlockSpec((H, H), lambda i: (0, 0)),
            pl.BlockSpec((H, H), lambda i: (0, 0)),
            pl.BlockSpec((H, H), lambda i: (0, 0)),
            pl.BlockSpec((H, H), lambda i: (0, 0)),
            pl.BlockSpec((1, H), lambda i: (0, 0)),
            pl.BlockSpec((1, H), lambda i: (0, 0)),
            pl.BlockSpec((H, H), lambda i: (0, 0)),
            pl.BlockSpec((1, H), lambda i: (0, 0)),
        ],
        out_specs=[
            pl.BlockSpec((BE, H), lambda i: (i, 0)),
            pl.BlockSpec((8, H), lambda i: (0, 0)),
        ],
        out_shape=[
            jax.ShapeDtypeStruct((e_num, H), jnp.float32),
            jax.ShapeDtypeStruct((8, H), jnp.float32),
        ],
    )(ea0, src, tgt, w_ea, w_src, w_tgt, w_ds, w_cos, b1, w2, b2)


def _tc_omega2(y, stats, gamma, beta, pw1, pb1c, pw2r, pb2):
    def k(y_ref, st_ref, g_ref, b_ref, w1_ref, b1_ref, w2_ref, b2_ref,
          om_ref):
        st = st_ref[...]
        mu = st[0:1] * (1.0 / E)
        var = st[1:2] * (1.0 / E) - mu * mu
        ea1 = _relu(g_ref[...] * (y_ref[...] - mu) * lax.rsqrt(var + BN_EPS)
                    + b_ref[...])
        hT = _relu(_dotg_t(w1_ref[...], ea1) + b1_ref[...])
        om = _dot(w2_ref[...], hT) + b2_ref[...]
        om_ref[...] = om[None]

    g = y.shape[0] // BE
    return pl.pallas_call(
        k,
        grid=(g,),
        in_specs=[
            pl.BlockSpec((BE, H), lambda i: (i, 0)),
            pl.BlockSpec((8, H), lambda i: (0, 0)),
            pl.BlockSpec((1, H), lambda i: (0, 0)),
            pl.BlockSpec((1, H), lambda i: (0, 0)),
            pl.BlockSpec((H, 32), lambda i: (0, 0)),
            pl.BlockSpec((32, 1), lambda i: (0, 0)),
            pl.BlockSpec((1, 32), lambda i: (0, 0)),
            pl.BlockSpec((1, 1), lambda i: (0, 0)),
        ],
        out_specs=pl.BlockSpec((1, 1, BE), lambda i: (i, 0, 0)),
        out_shape=jax.ShapeDtypeStruct((g, 1, BE), jnp.float32),
    )(y, stats, gamma, beta, pw1, pb1c, pw2r, pb2)


def kernel(x, edge_index, edge_attr, params):
    row = edge_index[0]
    col = edge_index[1]
    p = params
    b0, b1 = p['blocks'][0], p['blocks'][1]

    def r2(v):
        return v.reshape(1, -1)

    zeros_tile = jnp.zeros((ROWS_PER_TILE, H), jnp.float32)

    x0, t1 = _tc_node_encode(x, p['np_w'], r2(p['np_b']),
                             b0['pn_w'], r2(b0['pn_b']))
    ea0, om1 = _tc_edge_enc_om(edge_attr, p['ep_w'], r2(p['ep_b']),
                               b0['pe_w1'], b0['pe_b1'].reshape(32, 1),
                               b0['pe_w2'].reshape(1, 32),
                               b0['pe_b2'].reshape(1, 1))

    p1 = _sc_msg_scatter(t1, om1.reshape(E), row, col, zeros_tile)
    s1, st1 = _tc_sum_stats(x0, p1)
    xn1, t2 = _tc_bn_relu_node(s1, st1, r2(b0['bn_ng']), r2(b0['bn_nb']),
                               b1['pn_w'], r2(b1['pn_b']))

    E0 = 128000
    E1 = E - E0
    ee_w1 = b0['ee_w1']
    mlp_w = (ee_w1[0:H], ee_w1[H:2 * H], ee_w1[2 * H:3 * H],
             ee_w1[3 * H:4 * H], ee_w1[4 * H:4 * H + 1],
             r2(b0['ee_b1']), b0['ee_w2'], r2(b0['ee_b2']))
    src0, tgt0 = _sc_gather2(xn1, row, col, 0, E0)
    src1, tgt1 = _sc_gather2(xn1, row, col, E0, E1)
    y0, sta = _tc_edge_mlp(ea0, src0, tgt0, *mlp_w, 0, E0)
    y1, stb = _tc_edge_mlp(ea0, src1, tgt1, *mlp_w, E0, E1)
    ste = sta + stb

    om_w2 = (b1['pe_w1'], b1['pe_b1'].reshape(32, 1),
             b1['pe_w2'].reshape(1, 32), b1['pe_b2'].reshape(1, 1))
    om2a = _tc_omega2(y0, ste, r2(b0['bn_eg']), r2(b0['bn_eb']), *om_w2)
    om2b = _tc_omega2(y1, ste, r2(b0['bn_eg']), r2(b0['bn_eb']), *om_w2)
    om2 = jnp.concatenate([om2a.reshape(E0), om2b.reshape(E1)])
    p2 = _sc_msg_scatter(t2, om2, row, col, zeros_tile)
    s2, st2 = _tc_sum_stats(xn1, p2)
    xn2 = _tc_bn_relu_node(s2, st2, r2(b1['bn_ng']), r2(b1['bn_nb']))
    return xn2


# speedup vs baseline: 1.2405x; 1.1187x over previous
import functools

import jax
import jax.numpy as jnp
from jax import lax
from jax.experimental import pallas as pl
from jax.experimental.pallas import tpu as pltpu
from jax.experimental.pallas import tpu_sc as plsc

N = 10000
E = 320000
H = 128
NC = 2
NS = 16
NW = NC * NS
PER_W = E // NW
C = 80
CH = PER_W // C
NPAD = 10240
ROWS_PER_TILE = NPAD // NS

BN_EPS = 1e-5

def _mk_mesh():
    return plsc.VectorSubcoreMesh(core_axis_name="c", subcore_axis_name="s",
                                  num_cores=NC, num_subcores=NS)


def _sc_msg_scatter(t, om, row, col, zeros_tile):
    @functools.partial(
        pl.kernel,
        out_type=jax.ShapeDtypeStruct((NC, NPAD, H), jnp.float32),
        mesh=_mk_mesh(),
        scratch_types=[
            pltpu.VMEM((C,), jnp.int32), pltpu.VMEM((C,), jnp.int32),
            pltpu.VMEM((C,), jnp.int32), pltpu.VMEM((C,), jnp.int32),
            pltpu.VMEM((C,), jnp.float32), pltpu.VMEM((C,), jnp.float32),
            pltpu.VMEM((C, H), jnp.float32), pltpu.VMEM((C, H), jnp.float32),
            pltpu.VMEM_SHARED((NPAD, H), jnp.float32),
            pltpu.SemaphoreType.DMA, pltpu.SemaphoreType.DMA,
            pltpu.SemaphoreType.DMA, pltpu.SemaphoreType.DMA,
        ],
    )
    def k(t_hbm, om_hbm, row_hbm, col_hbm, zero_hbm, p_hbm,
          rowv0, rowv1, colv0, colv1, omv0, omv1, rows0, rows1, acc,
          isem0, isem1, gsem0, gsem1):
        cid = lax.axis_index("c")
        sid = lax.axis_index("s")
        wid = sid * NC + cid
        base = wid * PER_W
        bufs = ((rowv0, colv0, omv0, rows0, isem0, gsem0),
                (rowv1, colv1, omv1, rows1, isem1, gsem1))

        pltpu.sync_copy(zero_hbm,
                        acc.at[pl.ds(sid * ROWS_PER_TILE, ROWS_PER_TILE)])

        def idx_start(g, b):
            off = base + g * C
            rowv, colv, omv, _, isem, _ = bufs[b]
            pltpu.async_copy(row_hbm.at[pl.ds(off, C)], rowv, isem)
            pltpu.async_copy(col_hbm.at[pl.ds(off, C)], colv, isem)
            pltpu.async_copy(om_hbm.at[pl.ds(off, C)], omv, isem)

        def idx_wait(b):
            rowv, colv, omv, _, isem, _ = bufs[b]
            pltpu.make_async_copy(row_hbm.at[pl.ds(0, C)], rowv, isem).wait()
            pltpu.make_async_copy(col_hbm.at[pl.ds(0, C)], colv, isem).wait()
            pltpu.make_async_copy(om_hbm.at[pl.ds(0, C)], omv, isem).wait()

        def gather_start(b):
            rowv, _, _, rows, _, gsem = bufs[b]
            pltpu.async_copy(t_hbm.at[rowv], rows, gsem)

        def gather_wait(b):
            rowv, _, _, rows, _, gsem = bufs[b]
            pltpu.make_async_copy(t_hbm.at[rowv], rows, gsem).wait()

        def scale(b):
            _, _, omv, rows, _, _ = bufs[b]

            def body(e16, carry):
                om16 = omv[pl.ds(e16 * 16, 16)]
                for l in range(16):
                    om_vec = lax.gather(
                        om16, jnp.full((16, 1), l, jnp.int32),
                        lax.GatherDimensionNumbers(offset_dims=(),
                                                   collapsed_slice_dims=(0,),
                                                   start_index_map=(0,)),
                        (1,), mode=lax.GatherScatterMode.PROMISE_IN_BOUNDS)
                    e = e16 * 16 + l
                    for j in range(8):
                        sl = pl.ds(j * 16, 16)
                        rows[e, sl] = rows[e, sl] * om_vec
                return carry

            lax.fori_loop(0, C // 16, body, 0)

        idx_start(0, 0)
        plsc.subcore_barrier()
        idx_wait(0)
        gather_start(0)
        idx_start(1, 1)

        @pl.loop(0, CH, step=2)
        def _outer(g0):
            for b in range(2):
                g = g0 + b

                @pl.when(g < CH)
                def _():
                    _, colv, _, rows, _, _ = bufs[b]
                    gather_wait(b)

                    @pl.when(g + 1 < CH)
                    def _():
                        idx_wait(1 - b)
                        gather_start(1 - b)

                    scale(b)
                    pltpu.sync_copy(rows, acc.at[colv], add=True)

                    @pl.when(g + 2 < CH)
                    def _():
                        idx_start(g + 2, b)

        plsc.subcore_barrier()
        pltpu.sync_copy(
            acc.at[pl.ds(sid * ROWS_PER_TILE, ROWS_PER_TILE)],
            p_hbm.at[cid].at[pl.ds(sid * ROWS_PER_TILE, ROWS_PER_TILE)])

    return k(t, om, row, col, zeros_tile)


def _sc_gather2(xn, row, col, e_off, e_num):
    per_w = e_num // NW
    n_ch = per_w // C

    @functools.partial(
        pl.kernel,
        out_type=(
            jax.ShapeDtypeStruct((e_num, H), jnp.float32),
            jax.ShapeDtypeStruct((e_num, H), jnp.float32),
        ),
        mesh=_mk_mesh(),
        scratch_types=[
            pltpu.VMEM((C,), jnp.int32), pltpu.VMEM((C,), jnp.int32),
            pltpu.VMEM((C,), jnp.int32), pltpu.VMEM((C,), jnp.int32),
            pltpu.VMEM((C, H), jnp.float32), pltpu.VMEM((C, H), jnp.float32),
            pltpu.VMEM((C, H), jnp.float32), pltpu.VMEM((C, H), jnp.float32),
            pltpu.SemaphoreType.DMA, pltpu.SemaphoreType.DMA,
            pltpu.SemaphoreType.DMA, pltpu.SemaphoreType.DMA,
        ],
    )
    def k(xn_hbm, row_hbm, col_hbm, src_hbm, tgt_hbm,
          rowv0, rowv1, colv0, colv1, sb0, sb1, tb0, tb1,
          isem0, isem1, gsem0, gsem1):
        wid = lax.axis_index("s") * NC + lax.axis_index("c")
        base = wid * per_w

        bufs = ((rowv0, colv0, sb0, tb0, isem0, gsem0),
                (rowv1, colv1, sb1, tb1, isem1, gsem1))

        def idx_start(g, b):
            off = e_off + base + g * C
            rowv, colv, _, _, isem, _ = bufs[b]
            pltpu.async_copy(row_hbm.at[pl.ds(off, C)], rowv, isem)
            pltpu.async_copy(col_hbm.at[pl.ds(off, C)], colv, isem)

        def idx_wait(b):
            rowv, colv, _, _, isem, _ = bufs[b]
            pltpu.make_async_copy(row_hbm.at[pl.ds(0, C)], rowv, isem).wait()
            pltpu.make_async_copy(col_hbm.at[pl.ds(0, C)], colv, isem).wait()

        def gather_start(b):
            rowv, colv, sb, tb, _, gsem = bufs[b]
            pltpu.async_copy(xn_hbm.at[rowv], sb, gsem)
            pltpu.async_copy(xn_hbm.at[colv], tb, gsem)

        def gather_wait(b):
            rowv, colv, sb, tb, _, gsem = bufs[b]
            pltpu.make_async_copy(xn_hbm.at[rowv], sb, gsem).wait()
            pltpu.make_async_copy(xn_hbm.at[colv], tb, gsem).wait()

        idx_start(0, 0)
        idx_wait(0)
        gather_start(0)
        idx_start(1, 1)

        @pl.loop(0, n_ch, step=2)
        def _outer(g0):
            for b in range(2):
                g = g0 + b

                @pl.when(g < n_ch)
                def _():
                    _, _, sb, tb, _, _ = bufs[b]
                    gather_wait(b)

                    @pl.when(g + 1 < n_ch)
                    def _():
                        idx_wait(1 - b)
                        gather_start(1 - b)

                    @pl.when(g + 2 < n_ch)
                    def _():
                        idx_start(g + 2, b)

                    off = base + g * C
                    pltpu.sync_copy(sb, src_hbm.at[pl.ds(off, C)])
                    pltpu.sync_copy(tb, tgt_hbm.at[pl.ds(off, C)])

    return k(xn, row, col)


BN_TILE = 2000
BE = 4000


def _relu(v):
    return jnp.maximum(v, 0.0)


def _dot(a, b):
    return jnp.dot(a, b, preferred_element_type=jnp.float32)


def _tc_node_encode(x, npw, npb, pnw, pnb):
    def k(x_ref, npw_ref, npb_ref, pnw_ref, pnb_ref, x0_ref, t1_ref):
        x0 = _relu(_dot(x_ref[...], npw_ref[...]) + npb_ref[...])
        x0_ref[...] = x0
        t1_ref[...] = _relu(_dot(x0, pnw_ref[...]) + pnb_ref[...])

    g = N // BN_TILE
    return pl.pallas_call(
        k,
        grid=(g,),
        in_specs=[
            pl.BlockSpec((BN_TILE, H), lambda i: (i, 0)),
            pl.BlockSpec((H, H), lambda i: (0, 0)),
            pl.BlockSpec((1, H), lambda i: (0, 0)),
            pl.BlockSpec((H, H), lambda i: (0, 0)),
            pl.BlockSpec((1, H), lambda i: (0, 0)),
        ],
        out_specs=[
            pl.BlockSpec((BN_TILE, H), lambda i: (i, 0)),
            pl.BlockSpec((BN_TILE, H), lambda i: (i, 0)),
        ],
        out_shape=[
            jax.ShapeDtypeStruct((N, H), jnp.float32),
            jax.ShapeDtypeStruct((N, H), jnp.float32),
        ],
    )(x, npw, npb, pnw, pnb)


def _tc_edge_enc_om(edge_attr, epw, epb, pw1, pb1c, pw2r, pb2):
    def k(ea_ref, w_ref, b_ref, w1_ref, b1_ref, w2_ref, b2_ref,
          out_ref, om_ref):
        ea0 = _relu(_dot(ea_ref[...], w_ref[...]) + b_ref[...])
        out_ref[...] = ea0
        hT = _relu(_dotg_t(w1_ref[...], ea0) + b1_ref[...])
        om = _dot(w2_ref[...], hT) + b2_ref[...]
        om_ref[...] = om[None]

    g = E // BE
    d_edge = edge_attr.shape[1]
    return pl.pallas_call(
        k,
        grid=(g,),
        in_specs=[
            pl.BlockSpec((BE, d_edge), lambda i: (i, 0)),
            pl.BlockSpec((d_edge, H), lambda i: (0, 0)),
            pl.BlockSpec((1, H), lambda i: (0, 0)),
            pl.BlockSpec((H, 32), lambda i: (0, 0)),
            pl.BlockSpec((32, 1), lambda i: (0, 0)),
            pl.BlockSpec((1, 32), lambda i: (0, 0)),
            pl.BlockSpec((1, 1), lambda i: (0, 0)),
        ],
        out_specs=[
            pl.BlockSpec((BE, H), lambda i: (i, 0)),
            pl.BlockSpec((1, 1, BE), lambda i: (i, 0, 0)),
        ],
        out_shape=[
            jax.ShapeDtypeStruct((E, H), jnp.float32),
            jax.ShapeDtypeStruct((g, 1, BE), jnp.float32),
        ],
    )(edge_attr, epw, epb, pw1, pb1c, pw2r, pb2)


def _dotg_t(a, b):
    return lax.dot_general(a, b, (((0,), (1,)), ((), ())),
                           preferred_element_type=jnp.float32)


def _tc_sum_stats(xin, partials):
    def k(x_ref, p_ref, s_ref, st_ref):
        s = x_ref[...] + p_ref[0] + p_ref[1]
        s_ref[...] = s
        ones_row = jnp.ones((1, BN_TILE), jnp.float32)
        upd = jnp.concatenate(
            [_dot(ones_row, s), _dot(ones_row, s * s),
             jnp.zeros((6, H), jnp.float32)], axis=0)

        @pl.when(pl.program_id(0) == 0)
        def _():
            st_ref[...] = jnp.zeros_like(st_ref)

        st_ref[...] += upd

    g = N // BN_TILE
    return pl.pallas_call(
        k,
        grid=(g,),
        in_specs=[
            pl.BlockSpec((BN_TILE, H), lambda i: (i, 0)),
            pl.BlockSpec((NC, BN_TILE, H), lambda i: (0, i, 0)),
        ],
        out_specs=[
            pl.BlockSpec((BN_TILE, H), lambda i: (i, 0)),
            pl.BlockSpec((8, H), lambda i: (0, 0)),
        ],
        out_shape=[
            jax.ShapeDtypeStruct((N, H), jnp.float32),
            jax.ShapeDtypeStruct((8, H), jnp.float32),
        ],
    )(xin, partials)


def _tc_bn_relu_node(s, stats, gamma, beta, pnw=None, pnb=None):
    with_t = pnw is not None

    def k(*refs):
        if with_t:
            s_ref, st_ref, g_ref, b_ref, w_ref, wb_ref, xn_ref, t_ref = refs
        else:
            s_ref, st_ref, g_ref, b_ref, xn_ref = refs
        st = st_ref[...]
        mu = st[0:1] * (1.0 / N)
        var = st[1:2] * (1.0 / N) - mu * mu
        xn = _relu(g_ref[...] * (s_ref[...] - mu) * lax.rsqrt(var + BN_EPS)
                   + b_ref[...])
        xn_ref[...] = xn
        if with_t:
            t_ref[...] = _relu(_dot(xn, w_ref[...]) + wb_ref[...])

    g = N // BN_TILE
    in_specs = [
        pl.BlockSpec((BN_TILE, H), lambda i: (i, 0)),
        pl.BlockSpec((8, H), lambda i: (0, 0)),
        pl.BlockSpec((1, H), lambda i: (0, 0)),
        pl.BlockSpec((1, H), lambda i: (0, 0)),
    ]
    args = [s, stats, gamma, beta]
    out_specs = [pl.BlockSpec((BN_TILE, H), lambda i: (i, 0))]
    out_shape = [jax.ShapeDtypeStruct((N, H), jnp.float32)]
    if with_t:
        in_specs += [pl.BlockSpec((H, H), lambda i: (0, 0)),
                     pl.BlockSpec((1, H), lambda i: (0, 0))]
        args += [pnw, pnb]
        out_specs.append(pl.BlockSpec((BN_TILE, H), lambda i: (i, 0)))
        out_shape.append(jax.ShapeDtypeStruct((N, H), jnp.float32))
    res = pl.pallas_call(
        k, grid=(g,), in_specs=in_specs, out_specs=out_specs,
        out_shape=out_shape,
    )(*args)
    return res if with_t else res[0]


def _tc_edge_mlp(ea0, src, tgt, w_ea, w_src, w_tgt, w_ds, w_cos, b1, w2, b2,
                 e_off, e_num):
    def k(ea_ref, s_ref, t_ref, wea_ref, wsrc_ref, wtgt_ref, wds_ref,
          wcos_ref, b1_ref, w2_ref, b2_ref, y_ref, st_ref):
        s = s_ref[...]
        t = t_ref[...]
        d = jnp.abs(s - t)
        ones_col = jnp.ones((H, 1), jnp.float32)
        sn2 = _dot(s * s, ones_col)
        tn2 = _dot(t * t, ones_col)
        st = _dot(s * t, ones_col)
        cos = st / jnp.maximum(jnp.sqrt(sn2 * tn2), 1e-8)
        h = _relu(_dot(ea_ref[...], wea_ref[...]) + _dot(s, wsrc_ref[...])
                  + _dot(t, wtgt_ref[...]) + _dot(d, wds_ref[...])
                  + cos * wcos_ref[...] + b1_ref[...])
        y = _dot(h, w2_ref[...]) + b2_ref[...]
        y_ref[...] = y
        ones_row = jnp.ones((1, BE), jnp.float32)
        upd = jnp.concatenate(
            [_dot(ones_row, y), _dot(ones_row, y * y),
             jnp.zeros((6, H), jnp.float32)], axis=0)

        @pl.when(pl.program_id(0) == 0)
        def _():
            st_ref[...] = jnp.zeros_like(st_ref)

        st_ref[...] += upd

    g = e_num // BE
    blk0 = e_off // BE
    return pl.pallas_call(
        k,
        grid=(g,),
        in_specs=[
            pl.BlockSpec((BE, H), lambda i: (i + blk0, 0)),
            pl.BlockSpec((BE, H), lambda i: (i, 0)),
            pl.BlockSpec((BE, H), lambda i: (i, 0)),
            pl.BlockSpec((H, H), lambda i: (0, 0)),
            pl.BlockSpec((H, H), lambda i: (0, 0)),
            pl.BlockSpec((H, H), lambda i: (0, 0)),
            pl.BlockSpec((H, H), lambda i: (0, 0)),
            pl.BlockSpec((1, H), lambda i: (0, 0)),
            pl.BlockSpec((1, H), lambda i: (0, 0)),
            pl.BlockSpec((H, H), lambda i: (0, 0)),
            pl.BlockSpec((1, H), lambda i: (0, 0)),
        ],
        out_specs=[
            pl.BlockSpec((BE, H), lambda i: (i, 0)),
            pl.BlockSpec((8, H), lambda i: (0, 0)),
        ],
        out_shape=[
            jax.ShapeDtypeStruct((e_num, H), jnp.float32),
            jax.ShapeDtypeStruct((8, H), jnp.float32),
        ],
    )(ea0, src, tgt, w_ea, w_src, w_tgt, w_ds, w_cos, b1, w2, b2)


def _tc_omega2(y, stats, gamma, beta, pw1, pb1c, pw2r, pb2):
    def k(y_ref, st_ref, g_ref, b_ref, w1_ref, b1_ref, w2_ref, b2_ref,
          om_ref):
        st = st_ref[...]
        mu = st[0:1] * (1.0 / E)
        var = st[1:2] * (1.0 / E) - mu * mu
        ea1 = _relu(g_ref[...] * (y_ref[...] - mu) * lax.rsqrt(var + BN_EPS)
                    + b_ref[...])
        hT = _relu(_dotg_t(w1_ref[...], ea1) + b1_ref[...])
        om = _dot(w2_ref[...], hT) + b2_ref[...]
        om_ref[...] = om[None]

    g = y.shape[0] // BE
    return pl.pallas_call(
        k,
        grid=(g,),
        in_specs=[
            pl.BlockSpec((BE, H), lambda i: (i, 0)),
            pl.BlockSpec((8, H), lambda i: (0, 0)),
            pl.BlockSpec((1, H), lambda i: (0, 0)),
            pl.BlockSpec((1, H), lambda i: (0, 0)),
            pl.BlockSpec((H, 32), lambda i: (0, 0)),
            pl.BlockSpec((32, 1), lambda i: (0, 0)),
            pl.BlockSpec((1, 32), lambda i: (0, 0)),
            pl.BlockSpec((1, 1), lambda i: (0, 0)),
        ],
        out_specs=pl.BlockSpec((1, 1, BE), lambda i: (i, 0, 0)),
        out_shape=jax.ShapeDtypeStruct((g, 1, BE), jnp.float32),
    )(y, stats, gamma, beta, pw1, pb1c, pw2r, pb2)


def kernel(x, edge_index, edge_attr, params):
    row = edge_index[0]
    col = edge_index[1]
    p = params
    b0, b1 = p['blocks'][0], p['blocks'][1]

    def r2(v):
        return v.reshape(1, -1)

    zeros_tile = jnp.zeros((ROWS_PER_TILE, H), jnp.float32)

    x0, t1 = _tc_node_encode(x, p['np_w'], r2(p['np_b']),
                             b0['pn_w'], r2(b0['pn_b']))
    ea0, om1 = _tc_edge_enc_om(edge_attr, p['ep_w'], r2(p['ep_b']),
                               b0['pe_w1'], b0['pe_b1'].reshape(32, 1),
                               b0['pe_w2'].reshape(1, 32),
                               b0['pe_b2'].reshape(1, 1))

    p1 = _sc_msg_scatter(t1, om1.reshape(E), row, col, zeros_tile)
    s1, st1 = _tc_sum_stats(x0, p1)
    xn1, t2 = _tc_bn_relu_node(s1, st1, r2(b0['bn_ng']), r2(b0['bn_nb']),
                               b1['pn_w'], r2(b1['pn_b']))

    E0 = 128000
    E1 = E - E0
    ee_w1 = b0['ee_w1']
    mlp_w = (ee_w1[0:H], ee_w1[H:2 * H], ee_w1[2 * H:3 * H],
             ee_w1[3 * H:4 * H], ee_w1[4 * H:4 * H + 1],
             r2(b0['ee_b1']), b0['ee_w2'], r2(b0['ee_b2']))
    src0, tgt0 = _sc_gather2(xn1, row, col, 0, E0)
    src1, tgt1 = _sc_gather2(xn1, row, col, E0, E1)
    y0, sta = _tc_edge_mlp(ea0, src0, tgt0, *mlp_w, 0, E0)
    y1, stb = _tc_edge_mlp(ea0, src1, tgt1, *mlp_w, E0, E1)
    ste = sta + stb

    om_w2 = (b1['pe_w1'], b1['pe_b1'].reshape(32, 1),
             b1['pe_w2'].reshape(1, 32), b1['pe_b2'].reshape(1, 1))
    om2a = _tc_omega2(y0, ste, r2(b0['bn_eg']), r2(b0['bn_eb']), *om_w2)
    om2b = _tc_omega2(y1, ste, r2(b0['bn_eg']), r2(b0['bn_eb']), *om_w2)
    om2 = jnp.concatenate([om2a.reshape(E0), om2b.reshape(E1)])
    p2 = _sc_msg_scatter(t2, om2, row, col, zeros_tile)
    s2, st2 = _tc_sum_stats(xn1, p2)
    xn2 = _tc_bn_relu_node(s2, st2, r2(b1['bn_ng']), r2(b1['bn_nb']))
    return xn2


# speedup vs baseline: 1.2844x; 1.0354x over previous
import functools

import jax
import jax.numpy as jnp
from jax import lax
from jax.experimental import pallas as pl
from jax.experimental.pallas import tpu as pltpu
from jax.experimental.pallas import tpu_sc as plsc

N = 10000
E = 320000
H = 128
NC = 2
NS = 16
NW = NC * NS
PER_W = E // NW
C = 80
CH = PER_W // C
NPAD = 10240
ROWS_PER_TILE = NPAD // NS

BN_EPS = 1e-5

def _mk_mesh():
    return plsc.VectorSubcoreMesh(core_axis_name="c", subcore_axis_name="s",
                                  num_cores=NC, num_subcores=NS)


def _sc_msg_scatter(t, om, row, col, zeros_tile):
    @functools.partial(
        pl.kernel,
        out_type=jax.ShapeDtypeStruct((NC, NPAD, H), jnp.float32),
        mesh=_mk_mesh(),
        scratch_types=[
            pltpu.VMEM((C,), jnp.int32), pltpu.VMEM((C,), jnp.int32),
            pltpu.VMEM((C,), jnp.int32),
            pltpu.VMEM((C,), jnp.int32), pltpu.VMEM((C,), jnp.int32),
            pltpu.VMEM((C,), jnp.int32),
            pltpu.VMEM((C,), jnp.float32), pltpu.VMEM((C,), jnp.float32),
            pltpu.VMEM((C,), jnp.float32),
            pltpu.VMEM((C, H), jnp.float32), pltpu.VMEM((C, H), jnp.float32),
            pltpu.VMEM_SHARED((NPAD, H), jnp.float32),
            pltpu.SemaphoreType.DMA, pltpu.SemaphoreType.DMA,
            pltpu.SemaphoreType.DMA, pltpu.SemaphoreType.DMA,
            pltpu.SemaphoreType.DMA, pltpu.SemaphoreType.DMA,
            pltpu.SemaphoreType.DMA,
        ],
    )
    def k(t_hbm, om_hbm, row_hbm, col_hbm, zero_hbm, p_hbm,
          rowv0, rowv1, rowv2, colv0, colv1, colv2, omv0, omv1, omv2,
          rows0, rows1, acc,
          isem0, isem1, isem2, gsem0, gsem1, ssem0, ssem1):
        cid = lax.axis_index("c")
        sid = lax.axis_index("s")
        wid = sid * NC + cid
        base = wid * PER_W
        idxb = ((rowv0, colv0, omv0, isem0), (rowv1, colv1, omv1, isem1),
                (rowv2, colv2, omv2, isem2))
        rowsb = ((rows0, gsem0, ssem0), (rows1, gsem1, ssem1))

        pltpu.sync_copy(zero_hbm,
                        acc.at[pl.ds(sid * ROWS_PER_TILE, ROWS_PER_TILE)])

        def idx_start(g, i3):
            off = base + g * C
            rowv, colv, omv, isem = idxb[i3]
            pltpu.async_copy(row_hbm.at[pl.ds(off, C)], rowv, isem)
            pltpu.async_copy(col_hbm.at[pl.ds(off, C)], colv, isem)
            pltpu.async_copy(om_hbm.at[pl.ds(off, C)], omv, isem)

        def idx_wait(i3):
            rowv, colv, omv, isem = idxb[i3]
            pltpu.make_async_copy(row_hbm.at[pl.ds(0, C)], rowv, isem).wait()
            pltpu.make_async_copy(col_hbm.at[pl.ds(0, C)], colv, isem).wait()
            pltpu.make_async_copy(om_hbm.at[pl.ds(0, C)], omv, isem).wait()

        def gather_start(i3, r2):
            rowv = idxb[i3][0]
            rows, gsem, _ = rowsb[r2]
            pltpu.async_copy(t_hbm.at[rowv], rows, gsem)

        def gather_wait(i3, r2):
            rowv = idxb[i3][0]
            rows, gsem, _ = rowsb[r2]
            pltpu.make_async_copy(t_hbm.at[rowv], rows, gsem).wait()

        def scatter_start(i3, r2):
            colv = idxb[i3][1]
            rows, _, ssem = rowsb[r2]
            pltpu.async_copy(rows, acc.at[colv], ssem, add=True)

        def scatter_wait(i3, r2):
            colv = idxb[i3][1]
            rows, _, ssem = rowsb[r2]
            pltpu.make_async_copy(rows, acc.at[colv], ssem).wait()

        def scale(i3, r2):
            omv = idxb[i3][2]
            rows = rowsb[r2][0]

            def body(e16, carry):
                om16 = omv[pl.ds(e16 * 16, 16)]
                for l in range(16):
                    om_vec = lax.gather(
                        om16, jnp.full((16, 1), l, jnp.int32),
                        lax.GatherDimensionNumbers(offset_dims=(),
                                                   collapsed_slice_dims=(0,),
                                                   start_index_map=(0,)),
                        (1,), mode=lax.GatherScatterMode.PROMISE_IN_BOUNDS)
                    e = e16 * 16 + l
                    for j in range(8):
                        sl = pl.ds(j * 16, 16)
                        rows[e, sl] = rows[e, sl] * om_vec
                return carry

            lax.fori_loop(0, C // 16, body, 0)

        idx_start(0, 0)
        plsc.subcore_barrier()
        idx_wait(0)
        gather_start(0, 0)
        idx_start(1, 1)

        @pl.loop(0, CH, step=6)
        def _outer(g0):
            for kk in range(6):
                g = g0 + kk
                r = kk % 2
                i = kk % 3

                @pl.when(g < CH)
                def _():
                    gather_wait(i, r)

                    @pl.when(g + 1 < CH)
                    def _():
                        idx_wait((kk + 1) % 3)

                        @pl.when(g >= 1)
                        def _():
                            scatter_wait((kk + 2) % 3, 1 - r)

                        gather_start((kk + 1) % 3, 1 - r)

                    @pl.when(g + 2 < CH)
                    def _():
                        idx_start(g + 2, (kk + 2) % 3)

                    scale(i, r)
                    scatter_start(i, r)

        scatter_wait((CH - 2) % 3, (CH - 2) % 2)
        scatter_wait((CH - 1) % 3, (CH - 1) % 2)
        plsc.subcore_barrier()
        pltpu.sync_copy(
            acc.at[pl.ds(sid * ROWS_PER_TILE, ROWS_PER_TILE)],
            p_hbm.at[cid].at[pl.ds(sid * ROWS_PER_TILE, ROWS_PER_TILE)])

    return k(t, om, row, col, zeros_tile)


def _sc_gather2(xn, row, col, e_off, e_num):
    per_w = e_num // NW
    n_ch = per_w // C

    @functools.partial(
        pl.kernel,
        out_type=(
            jax.ShapeDtypeStruct((e_num, H), jnp.float32),
            jax.ShapeDtypeStruct((e_num, H), jnp.float32),
        ),
        mesh=_mk_mesh(),
        scratch_types=[
            pltpu.VMEM((C,), jnp.int32), pltpu.VMEM((C,), jnp.int32),
            pltpu.VMEM((C,), jnp.int32), pltpu.VMEM((C,), jnp.int32),
            pltpu.VMEM((C, H), jnp.float32), pltpu.VMEM((C, H), jnp.float32),
            pltpu.VMEM((C, H), jnp.float32), pltpu.VMEM((C, H), jnp.float32),
            pltpu.SemaphoreType.DMA, pltpu.SemaphoreType.DMA,
            pltpu.SemaphoreType.DMA, pltpu.SemaphoreType.DMA,
            pltpu.SemaphoreType.DMA, pltpu.SemaphoreType.DMA,
        ],
    )
    def k(xn_hbm, row_hbm, col_hbm, src_hbm, tgt_hbm,
          rowv0, rowv1, colv0, colv1, sb0, sb1, tb0, tb1,
          isem0, isem1, gsem0, gsem1, wsem0, wsem1):
        wid = lax.axis_index("s") * NC + lax.axis_index("c")
        base = wid * per_w
        wsems = (wsem0, wsem1)

        bufs = ((rowv0, colv0, sb0, tb0, isem0, gsem0),
                (rowv1, colv1, sb1, tb1, isem1, gsem1))

        def idx_start(g, b):
            off = e_off + base + g * C
            rowv, colv, _, _, isem, _ = bufs[b]
            pltpu.async_copy(row_hbm.at[pl.ds(off, C)], rowv, isem)
            pltpu.async_copy(col_hbm.at[pl.ds(off, C)], colv, isem)

        def idx_wait(b):
            rowv, colv, _, _, isem, _ = bufs[b]
            pltpu.make_async_copy(row_hbm.at[pl.ds(0, C)], rowv, isem).wait()
            pltpu.make_async_copy(col_hbm.at[pl.ds(0, C)], colv, isem).wait()

        def gather_start(b):
            rowv, colv, sb, tb, _, gsem = bufs[b]
            pltpu.async_copy(xn_hbm.at[rowv], sb, gsem)
            pltpu.async_copy(xn_hbm.at[colv], tb, gsem)

        def gather_wait(b):
            rowv, colv, sb, tb, _, gsem = bufs[b]
            pltpu.make_async_copy(xn_hbm.at[rowv], sb, gsem).wait()
            pltpu.make_async_copy(xn_hbm.at[colv], tb, gsem).wait()

        def write_start(g, b):
            off = base + g * C
            _, _, sb, tb, _, _ = bufs[b]
            wsem = wsems[b]
            pltpu.async_copy(sb, src_hbm.at[pl.ds(off, C)], wsem)
            pltpu.async_copy(tb, tgt_hbm.at[pl.ds(off, C)], wsem)

        def write_wait(b):
            _, _, sb, tb, _, _ = bufs[b]
            wsem = wsems[b]
            pltpu.make_async_copy(sb, src_hbm.at[pl.ds(0, C)], wsem).wait()
            pltpu.make_async_copy(tb, tgt_hbm.at[pl.ds(0, C)], wsem).wait()

        idx_start(0, 0)
        idx_wait(0)
        gather_start(0)
        idx_start(1, 1)

        @pl.loop(0, n_ch, step=2)
        def _outer(g0):
            for b in range(2):
                g = g0 + b

                @pl.when(g < n_ch)
                def _():
                    gather_wait(b)

                    @pl.when(g + 1 < n_ch)
                    def _():
                        idx_wait(1 - b)

                        @pl.when(g >= 1)
                        def _():
                            write_wait(1 - b)

                        gather_start(1 - b)

                    @pl.when(g + 2 < n_ch)
                    def _():
                        idx_start(g + 2, b)

                    write_start(g, b)

        write_wait((n_ch - 2) % 2)
        write_wait((n_ch - 1) % 2)

    return k(xn, row, col)


BN_TILE = 2000
BE = 4000


def _relu(v):
    return jnp.maximum(v, 0.0)


def _dot(a, b):
    return jnp.dot(a, b, preferred_element_type=jnp.float32)


def _tc_node_encode(x, npw, npb, pnw, pnb):
    def k(x_ref, npw_ref, npb_ref, pnw_ref, pnb_ref, x0_ref, t1_ref):
        x0 = _relu(_dot(x_ref[...], npw_ref[...]) + npb_ref[...])
        x0_ref[...] = x0
        t1_ref[...] = _relu(_dot(x0, pnw_ref[...]) + pnb_ref[...])

    g = N // BN_TILE
    return pl.pallas_call(
        k,
        grid=(g,),
        in_specs=[
            pl.BlockSpec((BN_TILE, H), lambda i: (i, 0)),
            pl.BlockSpec((H, H), lambda i: (0, 0)),
            pl.BlockSpec((1, H), lambda i: (0, 0)),
            pl.BlockSpec((H, H), lambda i: (0, 0)),
            pl.BlockSpec((1, H), lambda i: (0, 0)),
        ],
        out_specs=[
            pl.BlockSpec((BN_TILE, H), lambda i: (i, 0)),
            pl.BlockSpec((BN_TILE, H), lambda i: (i, 0)),
        ],
        out_shape=[
            jax.ShapeDtypeStruct((N, H), jnp.float32),
            jax.ShapeDtypeStruct((N, H), jnp.float32),
        ],
    )(x, npw, npb, pnw, pnb)


def _tc_edge_enc_om(edge_attr, epw, epb, pw1, pb1c, pw2r, pb2):
    def k(ea_ref, w_ref, b_ref, w1_ref, b1_ref, w2_ref, b2_ref,
          out_ref, om_ref):
        ea0 = _relu(_dot(ea_ref[...], w_ref[...]) + b_ref[...])
        out_ref[...] = ea0
        hT = _relu(_dotg_t(w1_ref[...], ea0) + b1_ref[...])
        om = _dot(w2_ref[...], hT) + b2_ref[...]
        om_ref[...] = om[None]

    g = E // BE
    d_edge = edge_attr.shape[1]
    return pl.pallas_call(
        k,
        grid=(g,),
        in_specs=[
            pl.BlockSpec((BE, d_edge), lambda i: (i, 0)),
            pl.BlockSpec((d_edge, H), lambda i: (0, 0)),
            pl.BlockSpec((1, H), lambda i: (0, 0)),
            pl.BlockSpec((H, 32), lambda i: (0, 0)),
            pl.BlockSpec((32, 1), lambda i: (0, 0)),
            pl.BlockSpec((1, 32), lambda i: (0, 0)),
            pl.BlockSpec((1, 1), lambda i: (0, 0)),
        ],
        out_specs=[
            pl.BlockSpec((BE, H), lambda i: (i, 0)),
            pl.BlockSpec((1, 1, BE), lambda i: (i, 0, 0)),
        ],
        out_shape=[
            jax.ShapeDtypeStruct((E, H), jnp.float32),
            jax.ShapeDtypeStruct((g, 1, BE), jnp.float32),
        ],
    )(edge_attr, epw, epb, pw1, pb1c, pw2r, pb2)


def _dotg_t(a, b):
    return lax.dot_general(a, b, (((0,), (1,)), ((), ())),
                           preferred_element_type=jnp.float32)


def _tc_sum_stats(xin, partials):
    def k(x_ref, p_ref, s_ref, st_ref):
        s = x_ref[...] + p_ref[0] + p_ref[1]
        s_ref[...] = s
        ones_row = jnp.ones((1, BN_TILE), jnp.float32)
        upd = jnp.concatenate(
            [_dot(ones_row, s), _dot(ones_row, s * s),
             jnp.zeros((6, H), jnp.float32)], axis=0)

        @pl.when(pl.program_id(0) == 0)
        def _():
            st_ref[...] = jnp.zeros_like(st_ref)

        st_ref[...] += upd

    g = N // BN_TILE
    return pl.pallas_call(
        k,
        grid=(g,),
        in_specs=[
            pl.BlockSpec((BN_TILE, H), lambda i: (i, 0)),
            pl.BlockSpec((NC, BN_TILE, H), lambda i: (0, i, 0)),
        ],
        out_specs=[
            pl.BlockSpec((BN_TILE, H), lambda i: (i, 0)),
            pl.BlockSpec((8, H), lambda i: (0, 0)),
        ],
        out_shape=[
            jax.ShapeDtypeStruct((N, H), jnp.float32),
            jax.ShapeDtypeStruct((8, H), jnp.float32),
        ],
    )(xin, partials)


def _tc_bn_relu_node(s, stats, gamma, beta, pnw=None, pnb=None):
    with_t = pnw is not None

    def k(*refs):
        if with_t:
            s_ref, st_ref, g_ref, b_ref, w_ref, wb_ref, xn_ref, t_ref = refs
        else:
            s_ref, st_ref, g_ref, b_ref, xn_ref = refs
        st = st_ref[...]
        mu = st[0:1] * (1.0 / N)
        var = st[1:2] * (1.0 / N) - mu * mu
        xn = _relu(g_ref[...] * (s_ref[...] - mu) * lax.rsqrt(var + BN_EPS)
                   + b_ref[...])
        xn_ref[...] = xn
        if with_t:
            t_ref[...] = _relu(_dot(xn, w_ref[...]) + wb_ref[...])

    g = N // BN_TILE
    in_specs = [
        pl.BlockSpec((BN_TILE, H), lambda i: (i, 0)),
        pl.BlockSpec((8, H), lambda i: (0, 0)),
        pl.BlockSpec((1, H), lambda i: (0, 0)),
        pl.BlockSpec((1, H), lambda i: (0, 0)),
    ]
    args = [s, stats, gamma, beta]
    out_specs = [pl.BlockSpec((BN_TILE, H), lambda i: (i, 0))]
    out_shape = [jax.ShapeDtypeStruct((N, H), jnp.float32)]
    if with_t:
        in_specs += [pl.BlockSpec((H, H), lambda i: (0, 0)),
                     pl.BlockSpec((1, H), lambda i: (0, 0))]
        args += [pnw, pnb]
        out_specs.append(pl.BlockSpec((BN_TILE, H), lambda i: (i, 0)))
        out_shape.append(jax.ShapeDtypeStruct((N, H), jnp.float32))
    res = pl.pallas_call(
        k, grid=(g,), in_specs=in_specs, out_specs=out_specs,
        out_shape=out_shape,
    )(*args)
    return res if with_t else res[0]


def _tc_edge_mlp(ea0, src, tgt, w_ea, w_src, w_tgt, w_ds, w_cos, b1, w2, b2,
                 e_off, e_num):
    def k(ea_ref, s_ref, t_ref, wea_ref, wsrc_ref, wtgt_ref, wds_ref,
          wcos_ref, b1_ref, w2_ref, b2_ref, y_ref, st_ref):
        s = s_ref[...]
        t = t_ref[...]
        d = jnp.abs(s - t)
        ones_col = jnp.ones((H, 1), jnp.float32)
        sn2 = _dot(s * s, ones_col)
        tn2 = _dot(t * t, ones_col)
        st = _dot(s * t, ones_col)
        cos = st / jnp.maximum(jnp.sqrt(sn2 * tn2), 1e-8)
        h = _relu(_dot(ea_ref[...], wea_ref[...]) + _dot(s, wsrc_ref[...])
                  + _dot(t, wtgt_ref[...]) + _dot(d, wds_ref[...])
                  + cos * wcos_ref[...] + b1_ref[...])
        y = _dot(h, w2_ref[...]) + b2_ref[...]
        y_ref[...] = y
        ones_row = jnp.ones((1, BE), jnp.float32)
        upd = jnp.concatenate(
            [_dot(ones_row, y), _dot(ones_row, y * y),
             jnp.zeros((6, H), jnp.float32)], axis=0)

        @pl.when(pl.program_id(0) == 0)
        def _():
            st_ref[...] = jnp.zeros_like(st_ref)

        st_ref[...] += upd

    g = e_num // BE
    blk0 = e_off // BE
    return pl.pallas_call(
        k,
        grid=(g,),
        in_specs=[
            pl.BlockSpec((BE, H), lambda i: (i + blk0, 0)),
            pl.BlockSpec((BE, H), lambda i: (i, 0)),
            pl.BlockSpec((BE, H), lambda i: (i, 0)),
            pl.BlockSpec((H, H), lambda i: (0, 0)),
            pl.BlockSpec((H, H), lambda i: (0, 0)),
            pl.BlockSpec((H, H), lambda i: (0, 0)),
            pl.BlockSpec((H, H), lambda i: (0, 0)),
            pl.BlockSpec((1, H), lambda i: (0, 0)),
            pl.BlockSpec((1, H), lambda i: (0, 0)),
            pl.BlockSpec((H, H), lambda i: (0, 0)),
            pl.BlockSpec((1, H), lambda i: (0, 0)),
        ],
        out_specs=[
            pl.BlockSpec((BE, H), lambda i: (i, 0)),
            pl.BlockSpec((8, H), lambda i: (0, 0)),
        ],
        out_shape=[
            jax.ShapeDtypeStruct((e_num, H), jnp.float32),
            jax.ShapeDtypeStruct((8, H), jnp.float32),
        ],
    )(ea0, src, tgt, w_ea, w_src, w_tgt, w_ds, w_cos, b1, w2, b2)


def _tc_omega2(y, stats, gamma, beta, pw1, pb1c, pw2r, pb2):
    def k(y_ref, st_ref, g_ref, b_ref, w1_ref, b1_ref, w2_ref, b2_ref,
          om_ref):
        st = st_ref[...]
        mu = st[0:1] * (1.0 / E)
        var = st[1:2] * (1.0 / E) - mu * mu
        ea1 = _relu(g_ref[...] * (y_ref[...] - mu) * lax.rsqrt(var + BN_EPS)
                    + b_ref[...])
        hT = _relu(_dotg_t(w1_ref[...], ea1) + b1_ref[...])
        om = _dot(w2_ref[...], hT) + b2_ref[...]
        om_ref[...] = om[None]

    g = y.shape[0] // BE
    return pl.pallas_call(
        k,
        grid=(g,),
        in_specs=[
            pl.BlockSpec((BE, H), lambda i: (i, 0)),
            pl.BlockSpec((8, H), lambda i: (0, 0)),
            pl.BlockSpec((1, H), lambda i: (0, 0)),
            pl.BlockSpec((1, H), lambda i: (0, 0)),
            pl.BlockSpec((H, 32), lambda i: (0, 0)),
            pl.BlockSpec((32, 1), lambda i: (0, 0)),
            pl.BlockSpec((1, 32), lambda i: (0, 0)),
            pl.BlockSpec((1, 1), lambda i: (0, 0)),
        ],
        out_specs=pl.BlockSpec((1, 1, BE), lambda i: (i, 0, 0)),
        out_shape=jax.ShapeDtypeStruct((g, 1, BE), jnp.float32),
    )(y, stats, gamma, beta, pw1, pb1c, pw2r, pb2)


def kernel(x, edge_index, edge_attr, params):
    row = edge_index[0]
    col = edge_index[1]
    p = params
    b0, b1 = p['blocks'][0], p['blocks'][1]

    def r2(v):
        return v.reshape(1, -1)

    zeros_tile = jnp.zeros((ROWS_PER_TILE, H), jnp.float32)

    x0, t1 = _tc_node_encode(x, p['np_w'], r2(p['np_b']),
                             b0['pn_w'], r2(b0['pn_b']))
    ea0, om1 = _tc_edge_enc_om(edge_attr, p['ep_w'], r2(p['ep_b']),
                               b0['pe_w1'], b0['pe_b1'].reshape(32, 1),
                               b0['pe_w2'].reshape(1, 32),
                               b0['pe_b2'].reshape(1, 1))

    p1 = _sc_msg_scatter(t1, om1.reshape(E), row, col, zeros_tile)
    s1, st1 = _tc_sum_stats(x0, p1)
    xn1, t2 = _tc_bn_relu_node(s1, st1, r2(b0['bn_ng']), r2(b0['bn_nb']),
                               b1['pn_w'], r2(b1['pn_b']))

    E0 = 128000
    E1 = E - E0
    ee_w1 = b0['ee_w1']
    mlp_w = (ee_w1[0:H], ee_w1[H:2 * H], ee_w1[2 * H:3 * H],
             ee_w1[3 * H:4 * H], ee_w1[4 * H:4 * H + 1],
             r2(b0['ee_b1']), b0['ee_w2'], r2(b0['ee_b2']))
    src0, tgt0 = _sc_gather2(xn1, row, col, 0, E0)
    src1, tgt1 = _sc_gather2(xn1, row, col, E0, E1)
    y0, sta = _tc_edge_mlp(ea0, src0, tgt0, *mlp_w, 0, E0)
    y1, stb = _tc_edge_mlp(ea0, src1, tgt1, *mlp_w, E0, E1)
    ste = sta + stb

    om_w2 = (b1['pe_w1'], b1['pe_b1'].reshape(32, 1),
             b1['pe_w2'].reshape(1, 32), b1['pe_b2'].reshape(1, 1))
    om2a = _tc_omega2(y0, ste, r2(b0['bn_eg']), r2(b0['bn_eb']), *om_w2)
    om2b = _tc_omega2(y1, ste, r2(b0['bn_eg']), r2(b0['bn_eb']), *om_w2)
    om2 = jnp.concatenate([om2a.reshape(E0), om2b.reshape(E1)])
    p2 = _sc_msg_scatter(t2, om2, row, col, zeros_tile)
    s2, st2 = _tc_sum_stats(xn1, p2)
    xn2 = _tc_bn_relu_node(s2, st2, r2(b1['bn_ng']), r2(b1['bn_nb']))
    return xn2
